# Initial kernel scaffold; baseline (speedup 1.0000x reference)
import functools

import jax
import jax.numpy as jnp
from jax import lax
from jax.experimental import pallas as pl
from jax.experimental.pallas import tpu as pltpu
from jax.experimental.pallas import tpu_sc as plsc

N = 10000
E = 160000
D = 256
HALF = 128
NC = 2
NS = 16
CHUNK = 40
EDGES_PER_SUB = E // NS
NCHUNKS = EDGES_PER_SUB // CHUNK
ROWS_PER_SUB = N // NS
_HI = jax.lax.Precision.HIGHEST


def _dot(a, b):
    return jnp.dot(a, b, precision=_HI, preferred_element_type=jnp.float32)


def _t1_body(pre_ref, w_ref, o0, o1, o2, o3):
    p = _dot(pre_ref[...], w_ref[...])
    o0[...] = p[:, 0:HALF]
    o1[...] = p[:, HALF:2 * HALF]
    o2[...] = p[:, 2 * HALF:3 * HALF]
    o3[...] = p[:, 3 * HALF:4 * HALF]


def _t1(pre_node, w12):
    R = 500
    return pl.pallas_call(
        _t1_body,
        grid=(N // R,),
        in_specs=[
            pl.BlockSpec((R, D), lambda i: (i, 0)),
            pl.BlockSpec((D, 2 * D), lambda i: (0, 0)),
        ],
        out_specs=[pl.BlockSpec((R, HALF), lambda i: (i, 0))] * 4,
        out_shape=[jax.ShapeDtypeStruct((N, HALF), jnp.float32)] * 4,
    )(pre_node, w12)


def _t2_body(ea_ref, w_ref, b_ref, o0, o1):
    p = _dot(ea_ref[...], w_ref[...]) + b_ref[...]
    o0[...] = p[:, 0:HALF]
    o1[...] = p[:, HALF:2 * HALF]


def _t2(edge_attr, w_c, be1):
    R = 1000
    return pl.pallas_call(
        _t2_body,
        grid=(E // R,),
        in_specs=[
            pl.BlockSpec((R, 16), lambda i: (i, 0)),
            pl.BlockSpec((16, D), lambda i: (0, 0)),
            pl.BlockSpec((1, D), lambda i: (0, 0)),
        ],
        out_specs=[pl.BlockSpec((R, HALF), lambda i: (i, 0))] * 2,
        out_shape=[jax.ShapeDtypeStruct((E, HALF), jnp.float32)] * 2,
    )(edge_attr, w_c, be1)


_MESH = plsc.VectorSubcoreMesh(
    core_axis_name="c", subcore_axis_name="s", num_cores=NC, num_subcores=NS)


@functools.partial(
    pl.kernel,
    out_type=(
        jax.ShapeDtypeStruct((N, HALF), jnp.float32),
        jax.ShapeDtypeStruct((N, HALF), jnp.float32),
        jax.ShapeDtypeStruct((N, 8), jnp.float32),
    ),
    mesh=_MESH,
    scratch_types=(
        pltpu.VMEM((CHUNK,), jnp.int32),
        pltpu.VMEM((CHUNK,), jnp.int32),
        pltpu.VMEM((CHUNK, HALF), jnp.float32),
        pltpu.VMEM((CHUNK, HALF), jnp.float32),
        pltpu.VMEM((CHUNK, HALF), jnp.float32),
        pltpu.VMEM((CHUNK, 8), jnp.float32),
        pltpu.VMEM_SHARED((N, HALF), jnp.float32),
        pltpu.VMEM_SHARED((N, 8), jnp.float32),
        pltpu.SemaphoreType.DMA,
        pltpu.SemaphoreType.DMA,
    ),
)
def _sc_edge_kernel(pa0, pa1, pb0, pb1, ea0, ea1, src_hbm, dst_hbm,
                    ones_hbm, zs_hbm, zc_hbm,
                    s0_out, s1_out, cnt_out,
                    idx_s, idx_d, buf_a, buf_b, buf_e, ones_v,
                    acc_sh, cnt_sh, sem_a, sem_b):
    c = lax.axis_index("c")
    s = lax.axis_index("s")
    row0 = s * ROWS_PER_SUB

    pltpu.sync_copy(zs_hbm, acc_sh.at[pl.ds(row0, ROWS_PER_SUB)])
    pltpu.sync_copy(zc_hbm, cnt_sh.at[pl.ds(row0, ROWS_PER_SUB)])
    pltpu.sync_copy(ones_hbm, ones_v)
    plsc.subcore_barrier()

    ebase = s * EDGES_PER_SUB

    def sweep(pa, pb, ea, with_counts):
        def chunk(j, carry):
            base = ebase + j * CHUNK
            pltpu.sync_copy(src_hbm.at[pl.ds(base, CHUNK)], idx_s)
            pltpu.sync_copy(dst_hbm.at[pl.ds(base, CHUNK)], idx_d)
            cp_a = pltpu.async_copy(pa.at[idx_s], buf_a, sem_a)
            cp_b = pltpu.async_copy(pb.at[idx_d], buf_b, sem_b)
            pltpu.sync_copy(ea.at[pl.ds(base, CHUNK), :], buf_e)
            cp_a.wait()
            cp_b.wait()

            def row(r, carry2):
                for k in range(HALF // 16):
                    sl = pl.ds(k * 16, 16)
                    v = buf_a[r, sl] + buf_b[r, sl] + buf_e[r, sl]
                    buf_a[r, sl] = jnp.maximum(v, 0.0)
                return carry2

            lax.fori_loop(0, CHUNK, row, 0)

            pltpu.sync_copy(buf_a, acc_sh.at[idx_d], add=True)
            if with_counts:
                pltpu.sync_copy(ones_v, cnt_sh.at[idx_d], add=True)
            return carry

        lax.fori_loop(0, NCHUNKS, chunk, 0)

    @pl.when(c == 0)
    def _():
        sweep(pa0, pb0, ea0, True)

    @pl.when(c == 1)
    def _():
        sweep(pa1, pb1, ea1, False)

    plsc.subcore_barrier()

    @pl.when(c == 0)
    def _():
        pltpu.sync_copy(acc_sh.at[pl.ds(row0, ROWS_PER_SUB)],
                        s0_out.at[pl.ds(row0, ROWS_PER_SUB)])
        pltpu.sync_copy(cnt_sh.at[pl.ds(row0, ROWS_PER_SUB)],
                        cnt_out.at[pl.ds(row0, ROWS_PER_SUB)])

    @pl.when(c == 1)
    def _():
        pltpu.sync_copy(acc_sh.at[pl.ds(row0, ROWS_PER_SUB)],
                        s1_out.at[pl.ds(row0, ROWS_PER_SUB)])


def _sc_edge(pa0, pa1, pb0, pb1, ea0, ea1, src, dst):
    ones = jnp.ones((CHUNK, 8), jnp.float32)
    zs = jnp.zeros((ROWS_PER_SUB, HALF), jnp.float32)
    zc = jnp.zeros((ROWS_PER_SUB, 8), jnp.float32)
    return _sc_edge_kernel(pa0, pa1, pb0, pb1, ea0, ea1, src, dst,
                           ones, zs, zc)


def _t3_body(s0_ref, s1_ref, cnt_ref, pre_ref, x_ref, we2_ref, be2_ref,
             wn1_ref, bn1_ref, wn2_ref, bn2_ref, g_ref, b_ref, o_ref):
    svals = jnp.concatenate([s0_ref[...], s1_ref[...]], axis=1)
    cnt = cnt_ref[...][:, 0:1]
    hmsg = _dot(svals, we2_ref[...]) + cnt * be2_ref[...]
    t = (_dot(pre_ref[...], wn1_ref[0:D, :])
         + _dot(hmsg, wn1_ref[D:2 * D, :])
         + _dot(x_ref[...], wn1_ref[2 * D:3 * D, :])
         + bn1_ref[...])
    t = jnp.maximum(t, 0.0)
    hn = _dot(t, wn2_ref[...]) + bn2_ref[...]
    y = pre_ref[...] + hn
    mu = jnp.mean(y, axis=1, keepdims=True)
    var = jnp.mean((y - mu) ** 2, axis=1, keepdims=True)
    o_ref[...] = (y - mu) * lax.rsqrt(var + 1e-5) * g_ref[...] + b_ref[...]


def _t3(s0, s1, cnt, pre_node, x, we2, be2, wn1, bn1, wn2, bn2, gamma, beta):
    R = 500
    full = lambda shape: pl.BlockSpec(shape, lambda i: tuple(0 for _ in shape))
    rowb = lambda shape: pl.BlockSpec(shape, lambda i: (i,) + tuple(0 for _ in shape[1:]))
    return pl.pallas_call(
        _t3_body,
        grid=(N // R,),
        in_specs=[
            rowb((R, HALF)), rowb((R, HALF)), rowb((R, 8)),
            rowb((R, D)), rowb((R, D)),
            full((D, D)), full((1, D)),
            full((3 * D, D)), full((1, D)),
            full((D, D)), full((1, D)),
            full((1, D)), full((1, D)),
        ],
        out_specs=rowb((R, D)),
        out_shape=jax.ShapeDtypeStruct((N, D), jnp.float32),
    )(s0, s1, cnt, pre_node, x, we2, be2, wn1, bn1, wn2, bn2, gamma, beta)


def kernel(x, pre_node, edge_index, edge_attr, We1, be1, We2, be2,
           Wn1, bn1, Wn2, bn2, gamma, beta):
    src = edge_index[0].astype(jnp.int32)
    dst = edge_index[1].astype(jnp.int32)
    w12 = jnp.concatenate([We1[0:D], We1[D:2 * D]], axis=1)
    pa0, pa1, pb0, pb1 = _t1(pre_node, w12)
    ea0, ea1 = _t2(edge_attr, We1[2 * D:], be1.reshape(1, D))
    s0, s1, cnt = _sc_edge(pa0, pa1, pb0, pb1, ea0, ea1, src, dst)
    return _t3(s0, s1, cnt, pre_node, x,
               We2, be2.reshape(1, D), Wn1, bn1.reshape(1, D),
               Wn2, bn2.reshape(1, D), gamma.reshape(1, D), beta.reshape(1, D))


# speedup vs baseline: 1.7263x; 1.7263x over previous
import functools

import jax
import jax.numpy as jnp
from jax import lax
from jax.experimental import pallas as pl
from jax.experimental.pallas import tpu as pltpu
from jax.experimental.pallas import tpu_sc as plsc

N = 10000
E = 160000
D = 256
HALF = 128
NC = 2
NS = 16
CHUNK = 40
EDGES_PER_SUB = E // NS
NCHUNKS = EDGES_PER_SUB // CHUNK
ROWS_PER_SUB = N // NS
_HI = jax.lax.Precision.HIGHEST


def _dot(a, b):
    return jnp.dot(a, b, precision=_HI, preferred_element_type=jnp.float32)


def _t1_body(pre_ref, w_ref, o0, o1, o2, o3):
    p = _dot(pre_ref[...], w_ref[...])
    o0[...] = p[:, 0:HALF]
    o1[...] = p[:, HALF:2 * HALF]
    o2[...] = p[:, 2 * HALF:3 * HALF]
    o3[...] = p[:, 3 * HALF:4 * HALF]


def _t1(pre_node, w12):
    R = 1000
    return pl.pallas_call(
        _t1_body,
        grid=(N // R,),
        in_specs=[
            pl.BlockSpec((R, D), lambda i: (i, 0)),
            pl.BlockSpec((D, 2 * D), lambda i: (0, 0)),
        ],
        out_specs=[pl.BlockSpec((R, HALF), lambda i: (i, 0))] * 4,
        out_shape=[jax.ShapeDtypeStruct((N, HALF), jnp.float32)] * 4,
    )(pre_node, w12)


def _t2_body(ea_ref, w_ref, b_ref, o0, o1):
    p = _dot(ea_ref[...], w_ref[...]) + b_ref[...]
    o0[...] = p[:, 0:HALF]
    o1[...] = p[:, HALF:2 * HALF]


def _t2(edge_attr, w_c, be1):
    R = 1000
    return pl.pallas_call(
        _t2_body,
        grid=(E // R,),
        in_specs=[
            pl.BlockSpec((R, 16), lambda i: (i, 0)),
            pl.BlockSpec((16, D), lambda i: (0, 0)),
            pl.BlockSpec((1, D), lambda i: (0, 0)),
        ],
        out_specs=[pl.BlockSpec((R, HALF), lambda i: (i, 0))] * 2,
        out_shape=[jax.ShapeDtypeStruct((E, HALF), jnp.float32)] * 2,
    )(edge_attr, w_c, be1)


@functools.cache
def _get_sc_kernel():
    mesh = plsc.VectorSubcoreMesh(
        core_axis_name="c", subcore_axis_name="s",
        num_cores=NC, num_subcores=NS)
    return functools.partial(
        pl.kernel,
        out_type=(
            jax.ShapeDtypeStruct((N, HALF), jnp.float32),
            jax.ShapeDtypeStruct((N, HALF), jnp.float32),
            jax.ShapeDtypeStruct((N, 8), jnp.float32),
        ),
        mesh=mesh,
        compiler_params=pltpu.CompilerParams(use_tc_tiling_on_sc=False),
        scratch_types=(
        pltpu.VMEM((CHUNK,), jnp.int32),
        pltpu.VMEM((CHUNK,), jnp.int32),
        pltpu.VMEM((CHUNK, HALF), jnp.float32),
        pltpu.VMEM((CHUNK, HALF), jnp.float32),
        pltpu.VMEM((CHUNK, HALF), jnp.float32),
        pltpu.VMEM((CHUNK, 8), jnp.float32),
        pltpu.VMEM_SHARED((N, HALF), jnp.float32),
        pltpu.VMEM_SHARED((N, 8), jnp.float32),
            pltpu.SemaphoreType.DMA,
            pltpu.SemaphoreType.DMA,
        ),
    )(_sc_edge_body)


def _sc_edge_body(pa0, pa1, pb0, pb1, ea0, ea1, src_hbm, dst_hbm,
                    ones_hbm, zs_hbm, zc_hbm,
                    s0_out, s1_out, cnt_out,
                    idx_s, idx_d, buf_a, buf_b, buf_e, ones_v,
                    acc_sh, cnt_sh, sem_a, sem_b):
    c = lax.axis_index("c")
    s = lax.axis_index("s")
    row0 = s * ROWS_PER_SUB

    pltpu.sync_copy(zs_hbm, acc_sh.at[pl.ds(row0, ROWS_PER_SUB)])
    pltpu.sync_copy(zc_hbm, cnt_sh.at[pl.ds(row0, ROWS_PER_SUB)])
    pltpu.sync_copy(ones_hbm, ones_v)
    plsc.subcore_barrier()

    ebase = s * EDGES_PER_SUB

    def sweep(pa, pb, ea, with_counts):
        def chunk(j, carry):
            base = ebase + j * CHUNK
            pltpu.sync_copy(src_hbm.at[pl.ds(base, CHUNK)], idx_s)
            pltpu.sync_copy(dst_hbm.at[pl.ds(base, CHUNK)], idx_d)
            cp_a = pltpu.async_copy(pa.at[idx_s], buf_a, sem_a)
            cp_b = pltpu.async_copy(pb.at[idx_d], buf_b, sem_b)
            pltpu.sync_copy(ea.at[pl.ds(base, CHUNK), :], buf_e)
            cp_a.wait()
            cp_b.wait()

            def row(r, carry2):
                for k in range(HALF // 16):
                    sl = pl.ds(k * 16, 16)
                    v = buf_a[r, sl] + buf_b[r, sl] + buf_e[r, sl]
                    buf_a[r, sl] = jnp.maximum(v, 0.0)
                return carry2

            lax.fori_loop(0, CHUNK, row, 0)

            pltpu.sync_copy(buf_a, acc_sh.at[idx_d], add=True)
            if with_counts:
                pltpu.sync_copy(ones_v, cnt_sh.at[idx_d], add=True)
            return carry

        lax.fori_loop(0, NCHUNKS, chunk, 0)

    @pl.when(c == 0)
    def _():
        sweep(pa0, pb0, ea0, True)

    @pl.when(c == 1)
    def _():
        sweep(pa1, pb1, ea1, False)

    plsc.subcore_barrier()

    @pl.when(c == 0)
    def _():
        pltpu.sync_copy(acc_sh.at[pl.ds(row0, ROWS_PER_SUB)],
                        s0_out.at[pl.ds(row0, ROWS_PER_SUB)])
        pltpu.sync_copy(cnt_sh.at[pl.ds(row0, ROWS_PER_SUB)],
                        cnt_out.at[pl.ds(row0, ROWS_PER_SUB)])

    @pl.when(c == 1)
    def _():
        pltpu.sync_copy(acc_sh.at[pl.ds(row0, ROWS_PER_SUB)],
                        s1_out.at[pl.ds(row0, ROWS_PER_SUB)])


def _sc_edge(pa0, pa1, pb0, pb1, ea0, ea1, src, dst):
    ones = jnp.ones((CHUNK, 8), jnp.float32)
    zs = jnp.zeros((ROWS_PER_SUB, HALF), jnp.float32)
    zc = jnp.zeros((ROWS_PER_SUB, 8), jnp.float32)
    return _get_sc_kernel()(pa0, pa1, pb0, pb1, ea0, ea1, src, dst,
                            ones, zs, zc)


def _t3_body(s0_ref, s1_ref, cnt_ref, pre_ref, x_ref, we2_ref, be2_ref,
             wn1_ref, bn1_ref, wn2_ref, bn2_ref, g_ref, b_ref, o_ref):
    svals = jnp.concatenate([s0_ref[...], s1_ref[...]], axis=1)
    cnt = cnt_ref[...][:, 0:1]
    hmsg = _dot(svals, we2_ref[...]) + cnt * be2_ref[...]
    t = (_dot(pre_ref[...], wn1_ref[0:D, :])
         + _dot(hmsg, wn1_ref[D:2 * D, :])
         + _dot(x_ref[...], wn1_ref[2 * D:3 * D, :])
         + bn1_ref[...])
    t = jnp.maximum(t, 0.0)
    hn = _dot(t, wn2_ref[...]) + bn2_ref[...]
    y = pre_ref[...] + hn
    mu = jnp.mean(y, axis=1, keepdims=True)
    var = jnp.mean((y - mu) ** 2, axis=1, keepdims=True)
    o_ref[...] = (y - mu) * lax.rsqrt(var + 1e-5) * g_ref[...] + b_ref[...]


def _t3(s0, s1, cnt, pre_node, x, we2, be2, wn1, bn1, wn2, bn2, gamma, beta):
    R = 1000
    full = lambda shape: pl.BlockSpec(shape, lambda i: tuple(0 for _ in shape))
    rowb = lambda shape: pl.BlockSpec(shape, lambda i: (i,) + tuple(0 for _ in shape[1:]))
    return pl.pallas_call(
        _t3_body,
        grid=(N // R,),
        in_specs=[
            rowb((R, HALF)), rowb((R, HALF)), rowb((R, 8)),
            rowb((R, D)), rowb((R, D)),
            full((D, D)), full((1, D)),
            full((3 * D, D)), full((1, D)),
            full((D, D)), full((1, D)),
            full((1, D)), full((1, D)),
        ],
        out_specs=rowb((R, D)),
        out_shape=jax.ShapeDtypeStruct((N, D), jnp.float32),
    )(s0, s1, cnt, pre_node, x, we2, be2, wn1, bn1, wn2, bn2, gamma, beta)


def kernel(x, pre_node, edge_index, edge_attr, We1, be1, We2, be2,
           Wn1, bn1, Wn2, bn2, gamma, beta):
    src = edge_index[0].astype(jnp.int32)
    dst = edge_index[1].astype(jnp.int32)
    w12 = jnp.concatenate([We1[0:D], We1[D:2 * D]], axis=1)
    pa0, pa1, pb0, pb1 = _t1(pre_node, w12)
    ea0, ea1 = _t2(edge_attr, We1[2 * D:], be1.reshape(1, D))
    s0, s1, cnt = _sc_edge(pa0, pa1, pb0, pb1, ea0, ea1, src, dst)
    return _t3(s0, s1, cnt, pre_node, x,
               We2, be2.reshape(1, D), Wn1, bn1.reshape(1, D),
               Wn2, bn2.reshape(1, D), gamma.reshape(1, D), beta.reshape(1, D))


# speedup vs baseline: 2.9572x; 1.7130x over previous
import functools

import jax
import jax.numpy as jnp
from jax import lax
from jax.experimental import pallas as pl
from jax.experimental.pallas import tpu as pltpu
from jax.experimental.pallas import tpu_sc as plsc

N = 10000
E = 160000
D = 256
HALF = 128
NC = 2
NS = 16
CHUNK = 40
EDGES_PER_SUB = E // NS
NCHUNKS = EDGES_PER_SUB // CHUNK
ROWS_PER_SUB = N // NS
_HI = jax.lax.Precision.HIGHEST


def _dot(a, b):
    return jnp.dot(a, b, precision=_HI, preferred_element_type=jnp.float32)


def _t1_body(pre_ref, w_ref, o0, o1, o2, o3):
    p = _dot(pre_ref[...], w_ref[...])
    o0[...] = p[:, 0:HALF]
    o1[...] = p[:, HALF:2 * HALF]
    o2[...] = p[:, 2 * HALF:3 * HALF]
    o3[...] = p[:, 3 * HALF:4 * HALF]


def _t1(pre_node, w12):
    R = 1000
    return pl.pallas_call(
        _t1_body,
        grid=(N // R,),
        in_specs=[
            pl.BlockSpec((R, D), lambda i: (i, 0)),
            pl.BlockSpec((D, 2 * D), lambda i: (0, 0)),
        ],
        out_specs=[pl.BlockSpec((R, HALF), lambda i: (i, 0))] * 4,
        out_shape=[jax.ShapeDtypeStruct((N, HALF), jnp.float32)] * 4,
    )(pre_node, w12)


def _t2_body(ea_ref, w_ref, b_ref, o0, o1):
    p = _dot(ea_ref[...], w_ref[...]) + b_ref[...]
    o0[...] = p[:, 0:HALF]
    o1[...] = p[:, HALF:2 * HALF]


def _t2(edge_attr, w_c, be1):
    R = 1000
    return pl.pallas_call(
        _t2_body,
        grid=(E // R,),
        in_specs=[
            pl.BlockSpec((R, 16), lambda i: (i, 0)),
            pl.BlockSpec((16, D), lambda i: (0, 0)),
            pl.BlockSpec((1, D), lambda i: (0, 0)),
        ],
        out_specs=[pl.BlockSpec((R, HALF), lambda i: (i, 0))] * 2,
        out_shape=[jax.ShapeDtypeStruct((E, HALF), jnp.float32)] * 2,
    )(edge_attr, w_c, be1)


NBUF = 2
NGROUPS = NCHUNKS // NBUF


@functools.cache
def _get_sc_kernel():
    mesh = plsc.VectorSubcoreMesh(
        core_axis_name="c", subcore_axis_name="s",
        num_cores=NC, num_subcores=NS)
    return functools.partial(
        pl.kernel,
        out_type=(
            jax.ShapeDtypeStruct((N, HALF), jnp.float32),
            jax.ShapeDtypeStruct((N, HALF), jnp.float32),
            jax.ShapeDtypeStruct((N, 8), jnp.float32),
        ),
        mesh=mesh,
        compiler_params=pltpu.CompilerParams(use_tc_tiling_on_sc=False),
        scratch_types=(
            pltpu.VMEM((CHUNK,), jnp.int32),
            pltpu.VMEM((CHUNK,), jnp.int32),
            pltpu.VMEM((CHUNK,), jnp.int32),
            pltpu.VMEM((CHUNK,), jnp.int32),
            pltpu.VMEM((CHUNK,), jnp.int32),
            pltpu.VMEM((CHUNK,), jnp.int32),
            pltpu.VMEM((CHUNK, HALF), jnp.float32),
            pltpu.VMEM((CHUNK, HALF), jnp.float32),
            pltpu.VMEM((CHUNK, HALF), jnp.float32),
            pltpu.VMEM((CHUNK, HALF), jnp.float32),
            pltpu.VMEM((CHUNK, HALF), jnp.float32),
            pltpu.VMEM((CHUNK, HALF), jnp.float32),
            pltpu.VMEM((CHUNK, HALF), jnp.float32),
            pltpu.VMEM((CHUNK, HALF), jnp.float32),
            pltpu.VMEM((CHUNK, 8), jnp.float32),
            pltpu.VMEM_SHARED((N, HALF), jnp.float32),
            pltpu.VMEM_SHARED((N, 8), jnp.float32),
            pltpu.SemaphoreType.DMA,
            pltpu.SemaphoreType.DMA,
            pltpu.SemaphoreType.DMA,
            pltpu.SemaphoreType.DMA,
            pltpu.SemaphoreType.DMA,
            pltpu.SemaphoreType.DMA,
            pltpu.SemaphoreType.DMA,
            pltpu.SemaphoreType.DMA,
            pltpu.SemaphoreType.DMA,
            pltpu.SemaphoreType.DMA,
            pltpu.SemaphoreType.DMA,
            pltpu.SemaphoreType.DMA,
            pltpu.SemaphoreType.DMA,
        ),
    )(_sc_edge_body)


def _sc_edge_body(pa0, pa1, pb0, pb1, ea0, ea1, src_hbm, dst_hbm,
                  ones_hbm, zs_hbm, zc_hbm,
                  s0_out, s1_out, cnt_out,
                  igs0, igs1, igd0, igd1, isd0, isd1,
                  buf_a0, buf_a1, buf_b0, buf_b1, buf_e0, buf_e1,
                  buf_z0, buf_z1, ones_v,
                  acc_sh, cnt_sh,
                  sem_a0, sem_a1, sem_b0, sem_b1, sem_e0, sem_e1,
                  sem_s0, sem_s1, sem_ig0, sem_ig1, sem_is0, sem_is1,
                  sem_c):
    c = lax.axis_index("c")
    s = lax.axis_index("s")
    row0 = s * ROWS_PER_SUB

    pltpu.sync_copy(zs_hbm, acc_sh.at[pl.ds(row0, ROWS_PER_SUB)])
    pltpu.sync_copy(zc_hbm, cnt_sh.at[pl.ds(row0, ROWS_PER_SUB)])
    pltpu.sync_copy(ones_hbm, ones_v)
    plsc.subcore_barrier()

    ebase = s * EDGES_PER_SUB
    slots = ((igs0, igd0, isd0, buf_a0, buf_b0, buf_e0, buf_z0,
              sem_a0, sem_b0, sem_e0, sem_s0, sem_ig0, sem_is0),
             (igs1, igd1, isd1, buf_a1, buf_b1, buf_e1, buf_z1,
              sem_a1, sem_b1, sem_e1, sem_s1, sem_ig1, sem_is1))

    def sweep(pa, pb, ea, with_counts):
        def issue_gidx(j, b):
            gs, gd, sd, ba, bb, be, bz, sa, sb, se, ss, sig, sis = slots[b]
            pltpu.async_copy(src_hbm.at[pl.ds(ebase + j * CHUNK, CHUNK)],
                             gs, sig)
            pltpu.async_copy(dst_hbm.at[pl.ds(ebase + j * CHUNK, CHUNK)],
                             gd, sig)

        def wait_gidx(j, b):
            gs, gd, sd, ba, bb, be, bz, sa, sb, se, ss, sig, sis = slots[b]
            pltpu.make_async_copy(
                src_hbm.at[pl.ds(ebase + j * CHUNK, CHUNK)], gs, sig).wait()
            pltpu.make_async_copy(
                dst_hbm.at[pl.ds(ebase + j * CHUNK, CHUNK)], gd, sig).wait()

        def issue_gathers(j, b):
            gs, gd, sd, ba, bb, be, bz, sa, sb, se, ss, sig, sis = slots[b]
            pltpu.async_copy(pa.at[gs], ba, sa)
            pltpu.async_copy(pb.at[gd], bb, sb)
            pltpu.async_copy(ea.at[pl.ds(ebase + j * CHUNK, CHUNK), :],
                             be, se)

        def wait_gathers(j, b):
            gs, gd, sd, ba, bb, be, bz, sa, sb, se, ss, sig, sis = slots[b]
            pltpu.make_async_copy(pa.at[gs], ba, sa).wait()
            pltpu.make_async_copy(pb.at[gd], bb, sb).wait()
            pltpu.make_async_copy(
                ea.at[pl.ds(ebase + j * CHUNK, CHUNK), :], be, se).wait()

        def compute(b):
            gs, gd, sd, ba, bb, be, bz, *_ = slots[b]

            def rowfn(r, carry):
                for k in range(HALF // 16):
                    sl = pl.ds(k * 16, 16)
                    v = ba[r, sl] + bb[r, sl] + be[r, sl]
                    bz[r, sl] = jnp.maximum(v, 0.0)
                return carry

            lax.fori_loop(0, CHUNK, rowfn, 0)

        def issue_scatter(j, b):
            gs, gd, sd, ba, bb, be, bz, sa, sb, se, ss, sig, sis = slots[b]
            pltpu.make_async_copy(
                dst_hbm.at[pl.ds(ebase + j * CHUNK, CHUNK)], sd, sis).wait()
            pltpu.async_copy(bz, acc_sh.at[sd], ss, add=True)
            if with_counts:
                pltpu.async_copy(ones_v, cnt_sh.at[sd], sem_c, add=True)

        def wait_scatter(j, b):
            gs, gd, sd, ba, bb, be, bz, sa, sb, se, ss, sig, sis = slots[b]
            pltpu.make_async_copy(bz, acc_sh.at[sd], ss).wait()

        def issue_sidx(j, b):
            gs, gd, sd, ba, bb, be, bz, sa, sb, se, ss, sig, sis = slots[b]
            pltpu.async_copy(dst_hbm.at[pl.ds(ebase + j * CHUNK, CHUNK)],
                             sd, sis)

        def step(j, b):
            wait_gathers(j, b)

            @pl.when(j + NBUF < NCHUNKS)
            def _():
                issue_gidx(j + NBUF, b)

            @pl.when(j >= NBUF)
            def _():
                wait_scatter(j - NBUF, b)

            issue_sidx(j, b)
            compute(b)
            issue_scatter(j, b)

            @pl.when(j + NBUF < NCHUNKS)
            def _():
                wait_gidx(j + NBUF, b)
                issue_gathers(j + NBUF, b)

        for b in range(NBUF):
            issue_gidx(b, b)
            wait_gidx(b, b)
            issue_gathers(b, b)

        def gbody(g, carry):
            for b in range(NBUF):
                step(g * NBUF + b, b)
            return carry

        lax.fori_loop(0, NGROUPS, gbody, 0)

        for b in range(NBUF):
            wait_scatter(NCHUNKS - NBUF + b, b)

        if with_counts:
            def drain(i, carry):
                gs, gd, sd = slots[0][0], slots[0][1], slots[0][2]
                pltpu.make_async_copy(ones_v, cnt_sh.at[sd], sem_c).wait()
                return carry

            lax.fori_loop(0, NCHUNKS, drain, 0)

    @pl.when(c == 0)
    def _():
        sweep(pa0, pb0, ea0, True)

    @pl.when(c == 1)
    def _():
        sweep(pa1, pb1, ea1, False)

    plsc.subcore_barrier()

    @pl.when(c == 0)
    def _():
        pltpu.sync_copy(acc_sh.at[pl.ds(row0, ROWS_PER_SUB)],
                        s0_out.at[pl.ds(row0, ROWS_PER_SUB)])
        pltpu.sync_copy(cnt_sh.at[pl.ds(row0, ROWS_PER_SUB)],
                        cnt_out.at[pl.ds(row0, ROWS_PER_SUB)])

    @pl.when(c == 1)
    def _():
        pltpu.sync_copy(acc_sh.at[pl.ds(row0, ROWS_PER_SUB)],
                        s1_out.at[pl.ds(row0, ROWS_PER_SUB)])


def _sc_edge(pa0, pa1, pb0, pb1, ea0, ea1, src, dst):
    ones = jnp.ones((CHUNK, 8), jnp.float32)
    zs = jnp.zeros((ROWS_PER_SUB, HALF), jnp.float32)
    zc = jnp.zeros((ROWS_PER_SUB, 8), jnp.float32)
    return _get_sc_kernel()(pa0, pa1, pb0, pb1, ea0, ea1, src, dst,
                            ones, zs, zc)


def _t3_body(s0_ref, s1_ref, cnt_ref, pre_ref, x_ref, we2_ref, be2_ref,
             wn1_ref, bn1_ref, wn2_ref, bn2_ref, g_ref, b_ref, o_ref):
    svals = jnp.concatenate([s0_ref[...], s1_ref[...]], axis=1)
    cnt = cnt_ref[...][:, 0:1]
    hmsg = _dot(svals, we2_ref[...]) + cnt * be2_ref[...]
    t = (_dot(pre_ref[...], wn1_ref[0:D, :])
         + _dot(hmsg, wn1_ref[D:2 * D, :])
         + _dot(x_ref[...], wn1_ref[2 * D:3 * D, :])
         + bn1_ref[...])
    t = jnp.maximum(t, 0.0)
    hn = _dot(t, wn2_ref[...]) + bn2_ref[...]
    y = pre_ref[...] + hn
    mu = jnp.mean(y, axis=1, keepdims=True)
    var = jnp.mean((y - mu) ** 2, axis=1, keepdims=True)
    o_ref[...] = (y - mu) * lax.rsqrt(var + 1e-5) * g_ref[...] + b_ref[...]


def _t3(s0, s1, cnt, pre_node, x, we2, be2, wn1, bn1, wn2, bn2, gamma, beta):
    R = 1000
    full = lambda shape: pl.BlockSpec(shape, lambda i: tuple(0 for _ in shape))
    rowb = lambda shape: pl.BlockSpec(shape, lambda i: (i,) + tuple(0 for _ in shape[1:]))
    return pl.pallas_call(
        _t3_body,
        grid=(N // R,),
        in_specs=[
            rowb((R, HALF)), rowb((R, HALF)), rowb((R, 8)),
            rowb((R, D)), rowb((R, D)),
            full((D, D)), full((1, D)),
            full((3 * D, D)), full((1, D)),
            full((D, D)), full((1, D)),
            full((1, D)), full((1, D)),
        ],
        out_specs=rowb((R, D)),
        out_shape=jax.ShapeDtypeStruct((N, D), jnp.float32),
    )(s0, s1, cnt, pre_node, x, we2, be2, wn1, bn1, wn2, bn2, gamma, beta)


def kernel(x, pre_node, edge_index, edge_attr, We1, be1, We2, be2,
           Wn1, bn1, Wn2, bn2, gamma, beta):
    src = edge_index[0].astype(jnp.int32)
    dst = edge_index[1].astype(jnp.int32)
    w12 = jnp.concatenate([We1[0:D], We1[D:2 * D]], axis=1)
    pa0, pa1, pb0, pb1 = _t1(pre_node, w12)
    ea0, ea1 = _t2(edge_attr, We1[2 * D:], be1.reshape(1, D))
    s0, s1, cnt = _sc_edge(pa0, pa1, pb0, pb1, ea0, ea1, src, dst)
    return _t3(s0, s1, cnt, pre_node, x,
               We2, be2.reshape(1, D), Wn1, bn1.reshape(1, D),
               Wn2, bn2.reshape(1, D), gamma.reshape(1, D), beta.reshape(1, D))


# speedup vs baseline: 3.5459x; 1.1991x over previous
import functools

import jax
import jax.numpy as jnp
from jax import lax
from jax.experimental import pallas as pl
from jax.experimental.pallas import tpu as pltpu
from jax.experimental.pallas import tpu_sc as plsc

N = 10000
E = 160000
D = 256
HALF = 128
NC = 2
NS = 16
CHUNK = 40
EDGES_PER_SUB = E // NS
NCHUNKS = EDGES_PER_SUB // CHUNK
ROWS_PER_SUB = N // NS
def _dot(a, b):
    return jnp.dot(a, b, preferred_element_type=jnp.float32)


def _t1_body(pre_ref, w_ref, o0, o1, o2, o3):
    p = _dot(pre_ref[...], w_ref[...])
    o0[...] = p[:, 0:HALF]
    o1[...] = p[:, HALF:2 * HALF]
    o2[...] = p[:, 2 * HALF:3 * HALF]
    o3[...] = p[:, 3 * HALF:4 * HALF]


def _t1(pre_node, w12):
    R = 1000
    return pl.pallas_call(
        _t1_body,
        grid=(N // R,),
        in_specs=[
            pl.BlockSpec((R, D), lambda i: (i, 0)),
            pl.BlockSpec((D, 2 * D), lambda i: (0, 0)),
        ],
        out_specs=[pl.BlockSpec((R, HALF), lambda i: (i, 0))] * 4,
        out_shape=[jax.ShapeDtypeStruct((N, HALF), jnp.float32)] * 4,
    )(pre_node, w12)


def _t2_body(ea_ref, w_ref, b_ref, o0, o1):
    p = _dot(ea_ref[...], w_ref[...]) + b_ref[...]
    o0[...] = p[:, 0:HALF]
    o1[...] = p[:, HALF:2 * HALF]


def _t2(edge_attr, w_c, be1):
    R = 1000
    return pl.pallas_call(
        _t2_body,
        grid=(E // R,),
        in_specs=[
            pl.BlockSpec((R, 16), lambda i: (i, 0)),
            pl.BlockSpec((16, D), lambda i: (0, 0)),
            pl.BlockSpec((1, D), lambda i: (0, 0)),
        ],
        out_specs=[pl.BlockSpec((R, HALF), lambda i: (i, 0))] * 2,
        out_shape=[jax.ShapeDtypeStruct((E, HALF), jnp.float32)] * 2,
    )(edge_attr, w_c, be1)


NBUF = 2
NGROUPS = NCHUNKS // NBUF


@functools.cache
def _get_sc_kernel():
    mesh = plsc.VectorSubcoreMesh(
        core_axis_name="c", subcore_axis_name="s",
        num_cores=NC, num_subcores=NS)
    return functools.partial(
        pl.kernel,
        out_type=(
            jax.ShapeDtypeStruct((N, HALF), jnp.float32),
            jax.ShapeDtypeStruct((N, HALF), jnp.float32),
            jax.ShapeDtypeStruct((N, 8), jnp.float32),
        ),
        mesh=mesh,
        compiler_params=pltpu.CompilerParams(use_tc_tiling_on_sc=False),
        scratch_types=(
            pltpu.VMEM((CHUNK,), jnp.int32),
            pltpu.VMEM((CHUNK,), jnp.int32),
            pltpu.VMEM((CHUNK,), jnp.int32),
            pltpu.VMEM((CHUNK,), jnp.int32),
            pltpu.VMEM((CHUNK,), jnp.int32),
            pltpu.VMEM((CHUNK,), jnp.int32),
            pltpu.VMEM((CHUNK, HALF), jnp.float32),
            pltpu.VMEM((CHUNK, HALF), jnp.float32),
            pltpu.VMEM((CHUNK, HALF), jnp.float32),
            pltpu.VMEM((CHUNK, HALF), jnp.float32),
            pltpu.VMEM((CHUNK, HALF), jnp.float32),
            pltpu.VMEM((CHUNK, HALF), jnp.float32),
            pltpu.VMEM((CHUNK, HALF), jnp.float32),
            pltpu.VMEM((CHUNK, HALF), jnp.float32),
            pltpu.VMEM((CHUNK, 8), jnp.float32),
            pltpu.VMEM_SHARED((N, HALF), jnp.float32),
            pltpu.VMEM_SHARED((N, 8), jnp.float32),
            pltpu.SemaphoreType.DMA,
            pltpu.SemaphoreType.DMA,
            pltpu.SemaphoreType.DMA,
            pltpu.SemaphoreType.DMA,
            pltpu.SemaphoreType.DMA,
            pltpu.SemaphoreType.DMA,
            pltpu.SemaphoreType.DMA,
            pltpu.SemaphoreType.DMA,
            pltpu.SemaphoreType.DMA,
            pltpu.SemaphoreType.DMA,
            pltpu.SemaphoreType.DMA,
            pltpu.SemaphoreType.DMA,
            pltpu.SemaphoreType.DMA,
        ),
    )(_sc_edge_body)


def _sc_edge_body(pa0, pa1, pb0, pb1, ea0, ea1, src_hbm, dst_hbm,
                  ones_hbm, zs_hbm, zc_hbm,
                  s0_out, s1_out, cnt_out,
                  igs0, igs1, igd0, igd1, isd0, isd1,
                  buf_a0, buf_a1, buf_b0, buf_b1, buf_e0, buf_e1,
                  buf_z0, buf_z1, ones_v,
                  acc_sh, cnt_sh,
                  sem_a0, sem_a1, sem_b0, sem_b1, sem_e0, sem_e1,
                  sem_s0, sem_s1, sem_ig0, sem_ig1, sem_is0, sem_is1,
                  sem_c):
    c = lax.axis_index("c")
    s = lax.axis_index("s")
    row0 = s * ROWS_PER_SUB

    pltpu.sync_copy(zs_hbm, acc_sh.at[pl.ds(row0, ROWS_PER_SUB)])
    pltpu.sync_copy(zc_hbm, cnt_sh.at[pl.ds(row0, ROWS_PER_SUB)])
    pltpu.sync_copy(ones_hbm, ones_v)
    plsc.subcore_barrier()

    ebase = s * EDGES_PER_SUB
    slots = ((igs0, igd0, isd0, buf_a0, buf_b0, buf_e0, buf_z0,
              sem_a0, sem_b0, sem_e0, sem_s0, sem_ig0, sem_is0),
             (igs1, igd1, isd1, buf_a1, buf_b1, buf_e1, buf_z1,
              sem_a1, sem_b1, sem_e1, sem_s1, sem_ig1, sem_is1))

    def sweep(pa, pb, ea, with_counts):
        def issue_gidx(j, b):
            gs, gd, sd, ba, bb, be, bz, sa, sb, se, ss, sig, sis = slots[b]
            pltpu.async_copy(src_hbm.at[pl.ds(ebase + j * CHUNK, CHUNK)],
                             gs, sig)
            pltpu.async_copy(dst_hbm.at[pl.ds(ebase + j * CHUNK, CHUNK)],
                             gd, sig)

        def wait_gidx(j, b):
            gs, gd, sd, ba, bb, be, bz, sa, sb, se, ss, sig, sis = slots[b]
            pltpu.make_async_copy(
                src_hbm.at[pl.ds(ebase + j * CHUNK, CHUNK)], gs, sig).wait()
            pltpu.make_async_copy(
                dst_hbm.at[pl.ds(ebase + j * CHUNK, CHUNK)], gd, sig).wait()

        def issue_gathers(j, b):
            gs, gd, sd, ba, bb, be, bz, sa, sb, se, ss, sig, sis = slots[b]
            pltpu.async_copy(pa.at[gs], ba, sa)
            pltpu.async_copy(pb.at[gd], bb, sb)
            pltpu.async_copy(ea.at[pl.ds(ebase + j * CHUNK, CHUNK), :],
                             be, se)

        def wait_gathers(j, b):
            gs, gd, sd, ba, bb, be, bz, sa, sb, se, ss, sig, sis = slots[b]
            pltpu.make_async_copy(pa.at[gs], ba, sa).wait()
            pltpu.make_async_copy(pb.at[gd], bb, sb).wait()
            pltpu.make_async_copy(
                ea.at[pl.ds(ebase + j * CHUNK, CHUNK), :], be, se).wait()

        def compute(b):
            gs, gd, sd, ba, bb, be, bz, *_ = slots[b]

            def rowfn(r, carry):
                for k in range(HALF // 16):
                    sl = pl.ds(k * 16, 16)
                    v = ba[r, sl] + bb[r, sl] + be[r, sl]
                    bz[r, sl] = jnp.maximum(v, 0.0)
                return carry

            lax.fori_loop(0, CHUNK, rowfn, 0)

        def issue_scatter(j, b):
            gs, gd, sd, ba, bb, be, bz, sa, sb, se, ss, sig, sis = slots[b]
            pltpu.make_async_copy(
                dst_hbm.at[pl.ds(ebase + j * CHUNK, CHUNK)], sd, sis).wait()
            pltpu.async_copy(bz, acc_sh.at[sd], ss, add=True)
            if with_counts:
                pltpu.async_copy(ones_v, cnt_sh.at[sd], sem_c, add=True)

        def wait_scatter(j, b):
            gs, gd, sd, ba, bb, be, bz, sa, sb, se, ss, sig, sis = slots[b]
            pltpu.make_async_copy(bz, acc_sh.at[sd], ss).wait()

        def issue_sidx(j, b):
            gs, gd, sd, ba, bb, be, bz, sa, sb, se, ss, sig, sis = slots[b]
            pltpu.async_copy(dst_hbm.at[pl.ds(ebase + j * CHUNK, CHUNK)],
                             sd, sis)

        def step(j, b):
            wait_gathers(j, b)

            @pl.when(j + NBUF < NCHUNKS)
            def _():
                issue_gidx(j + NBUF, b)

            @pl.when(j >= NBUF)
            def _():
                wait_scatter(j - NBUF, b)

            issue_sidx(j, b)
            compute(b)
            issue_scatter(j, b)

            @pl.when(j + NBUF < NCHUNKS)
            def _():
                wait_gidx(j + NBUF, b)
                issue_gathers(j + NBUF, b)

        for b in range(NBUF):
            issue_gidx(b, b)
            wait_gidx(b, b)
            issue_gathers(b, b)

        def gbody(g, carry):
            for b in range(NBUF):
                step(g * NBUF + b, b)
            return carry

        lax.fori_loop(0, NGROUPS, gbody, 0)

        for b in range(NBUF):
            wait_scatter(NCHUNKS - NBUF + b, b)

        if with_counts:
            def drain(i, carry):
                gs, gd, sd = slots[0][0], slots[0][1], slots[0][2]
                pltpu.make_async_copy(ones_v, cnt_sh.at[sd], sem_c).wait()
                return carry

            lax.fori_loop(0, NCHUNKS, drain, 0)

    @pl.when(c == 0)
    def _():
        sweep(pa0, pb0, ea0, True)

    @pl.when(c == 1)
    def _():
        sweep(pa1, pb1, ea1, False)

    plsc.subcore_barrier()

    @pl.when(c == 0)
    def _():
        pltpu.sync_copy(acc_sh.at[pl.ds(row0, ROWS_PER_SUB)],
                        s0_out.at[pl.ds(row0, ROWS_PER_SUB)])
        pltpu.sync_copy(cnt_sh.at[pl.ds(row0, ROWS_PER_SUB)],
                        cnt_out.at[pl.ds(row0, ROWS_PER_SUB)])

    @pl.when(c == 1)
    def _():
        pltpu.sync_copy(acc_sh.at[pl.ds(row0, ROWS_PER_SUB)],
                        s1_out.at[pl.ds(row0, ROWS_PER_SUB)])


def _sc_edge(pa0, pa1, pb0, pb1, ea0, ea1, src, dst):
    ones = jnp.ones((CHUNK, 8), jnp.float32)
    zs = jnp.zeros((ROWS_PER_SUB, HALF), jnp.float32)
    zc = jnp.zeros((ROWS_PER_SUB, 8), jnp.float32)
    return _get_sc_kernel()(pa0, pa1, pb0, pb1, ea0, ea1, src, dst,
                            ones, zs, zc)


def _t3_body(s0_ref, s1_ref, cnt_ref, pre_ref, x_ref, we2_ref, be2_ref,
             wn1_ref, bn1_ref, wn2_ref, bn2_ref, g_ref, b_ref, o_ref):
    svals = jnp.concatenate([s0_ref[...], s1_ref[...]], axis=1)
    cnt = cnt_ref[...][:, 0:1]
    hmsg = _dot(svals, we2_ref[...]) + cnt * be2_ref[...]
    t = (_dot(pre_ref[...], wn1_ref[0:D, :])
         + _dot(hmsg, wn1_ref[D:2 * D, :])
         + _dot(x_ref[...], wn1_ref[2 * D:3 * D, :])
         + bn1_ref[...])
    t = jnp.maximum(t, 0.0)
    hn = _dot(t, wn2_ref[...]) + bn2_ref[...]
    y = pre_ref[...] + hn
    mu = jnp.mean(y, axis=1, keepdims=True)
    var = jnp.mean((y - mu) ** 2, axis=1, keepdims=True)
    o_ref[...] = (y - mu) * lax.rsqrt(var + 1e-5) * g_ref[...] + b_ref[...]


def _t3(s0, s1, cnt, pre_node, x, we2, be2, wn1, bn1, wn2, bn2, gamma, beta):
    R = 1000
    full = lambda shape: pl.BlockSpec(shape, lambda i: tuple(0 for _ in shape))
    rowb = lambda shape: pl.BlockSpec(shape, lambda i: (i,) + tuple(0 for _ in shape[1:]))
    return pl.pallas_call(
        _t3_body,
        grid=(N // R,),
        in_specs=[
            rowb((R, HALF)), rowb((R, HALF)), rowb((R, 8)),
            rowb((R, D)), rowb((R, D)),
            full((D, D)), full((1, D)),
            full((3 * D, D)), full((1, D)),
            full((D, D)), full((1, D)),
            full((1, D)), full((1, D)),
        ],
        out_specs=rowb((R, D)),
        out_shape=jax.ShapeDtypeStruct((N, D), jnp.float32),
    )(s0, s1, cnt, pre_node, x, we2, be2, wn1, bn1, wn2, bn2, gamma, beta)


def kernel(x, pre_node, edge_index, edge_attr, We1, be1, We2, be2,
           Wn1, bn1, Wn2, bn2, gamma, beta):
    src = edge_index[0].astype(jnp.int32)
    dst = edge_index[1].astype(jnp.int32)
    w12 = jnp.concatenate([We1[0:D], We1[D:2 * D]], axis=1)
    pa0, pa1, pb0, pb1 = _t1(pre_node, w12)
    ea0, ea1 = _t2(edge_attr, We1[2 * D:], be1.reshape(1, D))
    s0, s1, cnt = _sc_edge(pa0, pa1, pb0, pb1, ea0, ea1, src, dst)
    return _t3(s0, s1, cnt, pre_node, x,
               We2, be2.reshape(1, D), Wn1, bn1.reshape(1, D),
               Wn2, bn2.reshape(1, D), gamma.reshape(1, D), beta.reshape(1, D))
